# Initial kernel scaffold; baseline (speedup 1.0000x reference)
import jax
import jax.numpy as jnp
from jax.experimental import pallas as pl


def kernel(var_val, var_type, length, W_emb, gamma, beta, W_ih, W_hh, b_ih, b_hh):
    raise NotImplementedError("write your pallas kernel here")


# speedup vs baseline: 4.8903x; 4.8903x over previous
import functools

import jax
import jax.numpy as jnp
from jax import lax
from jax.experimental import pallas as pl
from jax.experimental.pallas import tpu as pltpu, tpu_sc as plsc

B = 16
D = 64
V = 100000
TOT = 16384
LMAX = 1920
P = LMAX * B


def _sc_padded_gather():
    info = plsc.get_sparse_core_info()
    nc, ns, lanes = info.num_cores, info.num_subcores, info.num_lanes
    nw = nc * ns
    assert lanes == B, (lanes, B)
    assert LMAX % nw == 0
    t_w = LMAX // nw
    rows_w = P // nw
    n_chunk = 8
    c_sz = rows_w // n_chunk
    assert c_sz <= 128 and c_sz % 8 == 0

    mesh = plsc.VectorSubcoreMesh(core_axis_name="c", subcore_axis_name="s")

    @functools.partial(
        pl.kernel,
        mesh=mesh,
        compiler_params=pltpu.CompilerParams(
            needs_layout_passes=False, use_tc_tiling_on_sc=False),
        out_type=(
            jax.ShapeDtypeStruct((P, D), jnp.float32),
            jax.ShapeDtypeStruct((P,), jnp.float32),
        ),
        scratch_types=[
            pltpu.VMEM((TOT,), jnp.int32),
            pltpu.VMEM((TOT,), jnp.float32),
            pltpu.VMEM((B,), jnp.int32),
            pltpu.VMEM((B,), jnp.int32),
            pltpu.VMEM((rows_w,), jnp.int32),
            pltpu.VMEM((rows_w,), jnp.float32),
            pltpu.VMEM((rows_w, D), jnp.float32),
            pltpu.SemaphoreType.DMA,
        ],
    )
    def sc_kernel(vt_hbm, vv_hbm, off_hbm, len_hbm, emb_hbm,
                  xe_hbm, vvp_hbm,
                  vt_v, vv_v, off_v, len_v, gidx_v, vvp_v, rows_v, sem):
        w = lax.axis_index("s") * nc + lax.axis_index("c")
        pltpu.sync_copy(vt_hbm, vt_v)
        pltpu.sync_copy(vv_hbm, vv_v)
        pltpu.sync_copy(off_hbm, off_v)
        pltpu.sync_copy(len_hbm, len_v)
        offs = off_v[...]
        lens = len_v[...]
        t0 = w * t_w

        def body(jj, carry):
            tval = t0 + jj
            tok = offs + tval
            valid = tval < lens
            tokc = jnp.minimum(tok, TOT - 1)
            g = plsc.load_gather(vt_v, [tokc])
            g = jnp.where(valid, g, 0)
            gidx_v[pl.ds(jj * B, B)] = g
            vvl = plsc.load_gather(vv_v, [tokc])
            vvl = jnp.where(valid, vvl, jnp.float32(0.0))
            vvp_v[pl.ds(jj * B, B)] = vvl
            return carry

        lax.fori_loop(0, t_w, body, jnp.int32(0))

        copies = []
        for j in range(n_chunk):
            copies.append(pltpu.async_copy(
                emb_hbm.at[gidx_v.at[pl.ds(j * c_sz, c_sz)]],
                rows_v.at[pl.ds(j * c_sz, c_sz)],
                sem))
        for c in copies:
            c.wait()

        base = w * rows_w
        pltpu.sync_copy(rows_v, xe_hbm.at[pl.ds(base, rows_w)])
        pltpu.sync_copy(vvp_v, vvp_hbm.at[pl.ds(base, rows_w)])

    return sc_kernel


_CH = 256
_NCH = P // _CH


def _tc_body(xe_ref, vvp_ref, len_ref, g_ref, b_ref, wih_ref, whh_ref,
             bias_ref, out_ref, xp_ref):
    gamma = g_ref[...]
    beta = b_ref[...]
    wih = wih_ref[...]
    bias = bias_ref[...]

    def phase_a(i, carry):
        x = xe_ref[pl.ds(i * _CH, _CH), :]
        v = vvp_ref[pl.ds(i * _CH, _CH), :]
        mu = jnp.mean(x, axis=1, keepdims=True)
        d = x - mu
        var = jnp.mean(d * d, axis=1, keepdims=True)
        ln = d * lax.rsqrt(var + 1e-5) * gamma + beta
        xs = ln * v
        xp = lax.dot_general(xs, wih, (((1,), (1,)), ((), ())),
                             preferred_element_type=jnp.float32) + bias
        xp_ref[pl.ds(i * _CH, _CH), :] = xp
        return carry

    lax.fori_loop(0, _NCH, phase_a, jnp.int32(0))

    whh = whh_ref[...]
    lm1 = len_ref[...] - 1

    def phase_b(t, carry):
        h, out = carry
        xt = xp_ref[pl.ds(t * B, B), :]
        hn = jnp.tanh(xt + lax.dot_general(h, whh, (((1,), (1,)), ((), ())),
                                           preferred_element_type=jnp.float32))
        out = jnp.where(lm1 == t, hn, out)
        return (hn, out)

    zeros = jnp.zeros((B, D), jnp.float32)
    _, out = lax.fori_loop(0, LMAX, phase_b, (zeros, zeros))
    out_ref[...] = out


def _tc_call(xe, vvp, length, gamma, beta, W_ih, W_hh, bias):
    return pl.pallas_call(
        _tc_body,
        out_shape=jax.ShapeDtypeStruct((B, D), jnp.float32),
        scratch_shapes=[pltpu.VMEM((P, D), jnp.float32)],
    )(xe, vvp, length, gamma, beta, W_ih, W_hh, bias)


def kernel(var_val, var_type, length, W_emb, gamma, beta, W_ih, W_hh,
           b_ih, b_hh):
    length = length.astype(jnp.int32)
    offs = jnp.concatenate(
        [jnp.zeros((1,), jnp.int32), jnp.cumsum(length)[:-1]])
    sc = _sc_padded_gather()
    xe, vvp = sc(var_type.astype(jnp.int32), var_val, offs, length, W_emb)
    out = _tc_call(
        xe,
        vvp.reshape(P, 1),
        length.reshape(B, 1),
        gamma.reshape(1, D),
        beta.reshape(1, D),
        W_ih,
        W_hh,
        (b_ih + b_hh).reshape(1, D),
    )
    return out
